# baseline (speedup 1.0000x reference)
import functools

import jax
import jax.numpy as jnp
from jax import lax
from jax.experimental import pallas as pl
from jax.experimental.pallas import tpu as pltpu
from jax.experimental.pallas import tpu_sc as plsc

B = 16384
T = 512
L = 16

_info = plsc.get_sparse_core_info()
NC = _info.num_cores
NS = _info.num_subcores
NW = NC * NS
BPW = B // NW
CHUNK = 128
NCHUNK = BPW // CHUNK

_mesh = plsc.VectorSubcoreMesh(core_axis_name="c", subcore_axis_name="s")


@functools.partial(
    pl.kernel,
    mesh=_mesh,
    out_type=jax.ShapeDtypeStruct((B,), jnp.float32),
    scratch_types=[
        pltpu.VMEM((NCHUNK, CHUNK), jnp.int32),
        pltpu.VMEM((NCHUNK, CHUNK), jnp.float32),
        pltpu.SemaphoreType.DMA,
    ],
)
def _sc_gather(flat_hbm, s_hbm, out_hbm, idx_v, vals_v, sem):
    wid = lax.axis_index("s") * NC + lax.axis_index("c")
    base = wid * BPW

    for j in range(NCHUNK):
        pltpu.sync_copy(s_hbm.at[pl.ds(base + j * CHUNK, CHUNK)], idx_v.at[j])

    lane_t = lax.iota(jnp.int32, L) * T
    for j in range(NCHUNK):
        row = idx_v.at[j]
        for c in range(CHUNK // L):
            off = (base + j * CHUNK + c * L) * T
            row[pl.ds(c * L, L)] = row[pl.ds(c * L, L)] + (lane_t + off)

    copies = [
        pltpu.async_copy(flat_hbm.at[idx_v.at[j]], vals_v.at[j], sem)
        for j in range(NCHUNK)
    ]
    for cp in copies:
        cp.wait()

    for j in range(NCHUNK):
        pltpu.sync_copy(vals_v.at[j], out_hbm.at[pl.ds(base + j * CHUNK, CHUNK)])


def kernel(inputs, state):
    s = state.reshape(B).astype(jnp.int32)
    flat = inputs.reshape(B * T)
    out = _sc_gather(flat, s).reshape(B, 1)
    return (out, out)


# speedup vs baseline: 2.3710x; 2.3710x over previous
import functools

import jax
import jax.numpy as jnp
from jax import lax
from jax.experimental import pallas as pl
from jax.experimental.pallas import tpu as pltpu
from jax.experimental.pallas import tpu_sc as plsc

B = 16384
T = 512
L = 16

_info = plsc.get_sparse_core_info()
NC = _info.num_cores
NS = _info.num_subcores
NW = NC * NS
BPW = B // NW
CHUNK = 128
NCHUNK = BPW // CHUNK

_mesh = plsc.VectorSubcoreMesh(core_axis_name="c", subcore_axis_name="s")


@functools.partial(
    pl.kernel,
    mesh=_mesh,
    compiler_params=pltpu.CompilerParams(
        needs_layout_passes=False,
        skip_device_barrier=True,
        disable_bounds_checks=True,
        disable_semaphore_checks=True,
    ),
    out_type=[
        jax.ShapeDtypeStruct((NW, NCHUNK, CHUNK), jnp.float32),
        jax.ShapeDtypeStruct((NW, NCHUNK, CHUNK), jnp.float32),
    ],
    scratch_types=[
        pltpu.VMEM((NCHUNK, CHUNK), jnp.int32),
        pltpu.VMEM((NCHUNK, CHUNK), jnp.float32),
        [pltpu.SemaphoreType.DMA for _ in range(NCHUNK)],
    ],
)
def _sc_gather(flat_hbm, s_hbm, out0_hbm, out1_hbm, s_v, vals_v, sems):
    wid = lax.axis_index("s") * NC + lax.axis_index("c")

    pltpu.sync_copy(s_hbm.at[wid], s_v)

    lane = lax.iota(jnp.int32, L)
    lane_off = (lane >> 3) * 4096 + (lane & 7) * 128
    woff = wid * (BPW * T)
    copies = []
    for j in range(NCHUNK):
        s_row = s_v.at[j]
        for g in range(CHUNK // L):
            off = (j * CHUNK + g * L) * T
            sl = s_row[pl.ds(g * L, L)]
            s_row[pl.ds(g * L, L)] = (
                sl + (sl >> 7) * 896 + (lane_off + (woff + off))
            )
        copies.append(
            pltpu.async_copy(flat_hbm.at[s_v.at[j]], vals_v.at[j], sems[j])
        )
    for cp in copies:
        cp.wait()

    o0 = pltpu.async_copy(vals_v, out0_hbm.at[wid], sems[0])
    o1 = pltpu.async_copy(vals_v, out1_hbm.at[wid], sems[1])
    o0.wait()
    o1.wait()


def kernel(inputs, state):
    s = state.reshape(NW, NCHUNK, CHUNK).astype(jnp.int32)
    flat = (
        inputs.reshape(B // 8, 8, T // 128, 128)
        .transpose(0, 2, 1, 3)
        .reshape(B * T)
    )
    o0, o1 = _sc_gather(flat, s)
    return (o0.reshape(B, 1), o1.reshape(B, 1))


# speedup vs baseline: 2.3815x; 1.0045x over previous
import functools

import jax
import jax.numpy as jnp
from jax import lax
from jax.experimental import pallas as pl
from jax.experimental.pallas import tpu as pltpu
from jax.experimental.pallas import tpu_sc as plsc

B = 16384
T = 512
L = 16

_info = plsc.get_sparse_core_info()
NC = _info.num_cores
NS = _info.num_subcores
NW = NC * NS
BPW = B // NW
CHUNK = 128
NCHUNK = BPW // CHUNK

_mesh = plsc.VectorSubcoreMesh(core_axis_name="c", subcore_axis_name="s")


@functools.partial(
    pl.kernel,
    mesh=_mesh,
    compiler_params=pltpu.CompilerParams(needs_layout_passes=False),
    out_type=[
        jax.ShapeDtypeStruct((NW, NCHUNK, CHUNK), jnp.float32),
        jax.ShapeDtypeStruct((NW, NCHUNK, CHUNK), jnp.float32),
    ],
    scratch_types=[
        pltpu.VMEM((NCHUNK, CHUNK), jnp.int32),
        pltpu.VMEM((NCHUNK, CHUNK), jnp.float32),
        [pltpu.SemaphoreType.DMA for _ in range(NCHUNK)],
    ],
)
def _sc_gather(flat_hbm, s_hbm, out0_hbm, out1_hbm, s_v, vals_v, sems):
    wid = lax.axis_index("s") * NC + lax.axis_index("c")

    pltpu.sync_copy(s_hbm.at[wid], s_v)

    lane = lax.iota(jnp.int32, L)
    lane_off = (lane >> 3) * 4096 + (lane & 7) * 128
    woff = wid * (BPW * T)
    copies = []
    for j in range(NCHUNK):
        s_row = s_v.at[j]
        for g in range(CHUNK // L):
            off = (j * CHUNK + g * L) * T
            sl = s_row[pl.ds(g * L, L)]
            s_row[pl.ds(g * L, L)] = (
                sl + (sl >> 7) * 896 + (lane_off + (woff + off))
            )
        copies.append(
            pltpu.async_copy(flat_hbm.at[s_v.at[j]], vals_v.at[j], sems[j])
        )
    for cp in copies:
        cp.wait()

    o0 = pltpu.async_copy(vals_v, out0_hbm.at[wid], sems[0])
    o1 = pltpu.async_copy(vals_v, out1_hbm.at[wid], sems[1])
    o0.wait()
    o1.wait()


def kernel(inputs, state):
    s = state.reshape(NW, NCHUNK, CHUNK).astype(jnp.int32)
    flat = (
        inputs.reshape(B // 8, 8, T // 128, 128)
        .transpose(0, 2, 1, 3)
        .reshape(B * T)
    )
    o0, o1 = _sc_gather(flat, s)
    return (o0.reshape(B, 1), o1.reshape(B, 1))
